# baseline (speedup 1.0000x reference)
import functools

import jax
import jax.numpy as jnp
from jax.experimental import pallas as pl

_MARGIN = 1.0
_BN = 512


def _triplet_kernel(a_ref, p_ref, out_ref):
    a_t = a_ref[...].T
    p_t = p_ref[...].T
    _, b = a_t.shape

    an2 = jnp.sum(a_t * a_t, axis=0, keepdims=True)
    pn2 = jnp.sum(p_t * p_t, axis=0, keepdims=True)
    ap_dot = jnp.sum(a_t * p_t, axis=0, keepdims=True)
    p_n = (p_t * jax.lax.rsqrt(pn2)).astype(jnp.bfloat16)
    a_h = a_t.astype(jnp.bfloat16)

    eye = (jax.lax.broadcasted_iota(jnp.int32, (_BN, _BN), 0)
           == jax.lax.broadcasted_iota(jnp.int32, (_BN, _BN), 1))
    dims = (((0,), (0,)), ((), ()))

    acc = jnp.full((1, b), jnp.inf, jnp.float32)
    for j in range(b // _BN):
        lo, hi = j * _BN, (j + 1) * _BN
        tile = jax.lax.dot_general(p_n[:, lo:hi], a_h, dims,
                                   preferred_element_type=jnp.float32)
        m = jnp.min(tile, axis=0, keepdims=True)
        sub = jnp.where(eye, jnp.inf, tile[:, lo:hi])
        m_sub = jnp.min(sub, axis=0, keepdims=True)
        pieces = (([m[:, :lo]] if lo else []) + [m_sub]
                  + ([m[:, hi:]] if hi < b else []))
        m = jnp.concatenate(pieces, axis=1) if len(pieces) > 1 else m_sub
        acc = jnp.minimum(acc, m)

    an_dist = acc * jax.lax.rsqrt(an2)
    ap_dist = ap_dot / jnp.maximum(jnp.sqrt(an2) * jnp.sqrt(pn2), 1e-8)

    loss = jnp.sum(jax.nn.relu(_MARGIN + ap_dist - an_dist)) / b
    out_ref[...] = loss.reshape(1, 1)


@functools.partial(jax.jit, static_argnames=("interpret",))
def kernel(anchor, positive, interpret=False):
    out = pl.pallas_call(
        _triplet_kernel,
        out_shape=jax.ShapeDtypeStruct((1, 1), jnp.float32),
        interpret=interpret,
    )(anchor, positive)
    return out[0, 0]


# speedup vs baseline: 1.3221x; 1.3221x over previous
import functools

import jax
import jax.numpy as jnp
from jax.experimental import pallas as pl

_MARGIN = 1.0
_BN = 512


def _triplet_kernel(x_ref, out_ref):
    d2, b = x_ref.shape
    d = d2 // 2
    a_t = x_ref[:d, :]
    p_t = x_ref[d:, :]

    an2 = jnp.sum(a_t * a_t, axis=0, keepdims=True)
    pn2 = jnp.sum(p_t * p_t, axis=0, keepdims=True)
    ap_dot = jnp.sum(a_t * p_t, axis=0, keepdims=True)
    p_n = (p_t * jax.lax.rsqrt(pn2)).astype(jnp.bfloat16)
    a_h = a_t.astype(jnp.bfloat16)

    eye = (jax.lax.broadcasted_iota(jnp.int32, (_BN, _BN), 0)
           == jax.lax.broadcasted_iota(jnp.int32, (_BN, _BN), 1))
    dims = (((0,), (0,)), ((), ()))

    acc = jnp.full((1, b), jnp.inf, jnp.float32)
    for j in range(b // _BN):
        lo, hi = j * _BN, (j + 1) * _BN
        tile = jax.lax.dot_general(p_n[:, lo:hi], a_h, dims,
                                   preferred_element_type=jnp.float32)
        m = jnp.min(tile, axis=0, keepdims=True)
        sub = jnp.where(eye, jnp.inf, tile[:, lo:hi])
        m_sub = jnp.min(sub, axis=0, keepdims=True)
        pieces = (([m[:, :lo]] if lo else []) + [m_sub]
                  + ([m[:, hi:]] if hi < b else []))
        m = jnp.concatenate(pieces, axis=1) if len(pieces) > 1 else m_sub
        acc = jnp.minimum(acc, m)

    an_dist = acc * jax.lax.rsqrt(an2)
    ap_dist = ap_dot / jnp.maximum(jnp.sqrt(an2) * jnp.sqrt(pn2), 1e-8)

    loss = jnp.sum(jax.nn.relu(_MARGIN + ap_dist - an_dist)) / b
    out_ref[...] = loss.reshape(1, 1)


@functools.partial(jax.jit, static_argnames=("interpret",))
def kernel(anchor, positive, interpret=False):
    out = pl.pallas_call(
        _triplet_kernel,
        out_shape=jax.ShapeDtypeStruct((1, 1), jnp.float32),
        interpret=interpret,
    )(jnp.concatenate([anchor, positive], axis=1).T)
    return out[0, 0]


# speedup vs baseline: 1.5562x; 1.1771x over previous
import functools

import jax
import jax.numpy as jnp
from jax.experimental import pallas as pl

_MARGIN = 1.0
_BN = 512
_BIG = 1e30


def _tree_min8(tile):
    chunks = [tile[k:k + 8, :] for k in range(0, tile.shape[0], 8)]
    while len(chunks) > 1:
        nxt = [jnp.minimum(chunks[i], chunks[i + 1])
               for i in range(0, len(chunks) - 1, 2)]
        if len(chunks) % 2:
            nxt.append(chunks[-1])
        chunks = nxt
    return chunks[0]


def _triplet_kernel(a_ref, p_ref, out_ref):
    a_t = a_ref[...]
    p_t = p_ref[...]
    _, b = a_t.shape

    an2 = jnp.sum(a_t * a_t, axis=0, keepdims=True)
    pn2 = jnp.sum(p_t * p_t, axis=0, keepdims=True)
    ap_dot = jnp.sum(a_t * p_t, axis=0, keepdims=True)
    p_n = (p_t * jax.lax.rsqrt(pn2)).astype(jnp.bfloat16)
    a_h = a_t.astype(jnp.bfloat16)

    big_eye = jnp.where(
        jax.lax.broadcasted_iota(jnp.int32, (_BN, _BN), 0)
        == jax.lax.broadcasted_iota(jnp.int32, (_BN, _BN), 1),
        jnp.float32(_BIG), jnp.float32(0.0))
    dims = (((0,), (0,)), ((), ()))

    acc = jnp.full((8, b), jnp.inf, jnp.float32)
    for j in range(b // _BN):
        lo, hi = j * _BN, (j + 1) * _BN
        tile = jax.lax.dot_general(p_n[:, lo:hi], a_h, dims,
                                   preferred_element_type=jnp.float32)
        m8 = _tree_min8(tile)
        m_sub8 = _tree_min8(tile[:, lo:hi] + big_eye)
        pieces = (([m8[:, :lo]] if lo else []) + [m_sub8]
                  + ([m8[:, hi:]] if hi < b else []))
        m8 = jnp.concatenate(pieces, axis=1) if len(pieces) > 1 else m_sub8
        acc = jnp.minimum(acc, m8)

    an_min = jnp.min(acc, axis=0, keepdims=True)
    an_dist = an_min * jax.lax.rsqrt(an2)
    ap_dist = ap_dot / jnp.maximum(jnp.sqrt(an2) * jnp.sqrt(pn2), 1e-8)

    loss = jnp.sum(jax.nn.relu(_MARGIN + ap_dist - an_dist)) / b
    out_ref[...] = loss.reshape(1, 1)


@functools.partial(jax.jit, static_argnames=("interpret",))
def kernel(anchor, positive, interpret=False):
    out = pl.pallas_call(
        _triplet_kernel,
        out_shape=jax.ShapeDtypeStruct((1, 1), jnp.float32),
        interpret=interpret,
    )(anchor.T, positive.T)
    return out[0, 0]
